# baseline (speedup 1.0000x reference)
import functools

import jax
import jax.numpy as jnp
from jax.experimental import pallas as pl
from jax.experimental.pallas import tpu as pltpu

D = 1024
H = 4
DK = 64
DV = 64
M = 4
KC = 4
I = 2048
EPS = 1e-6
C = 64
LEV = 6
MH = M * H
PW = 1152
F3 = 3 * H * DK


def _f32dot(a, b):
    return jnp.dot(a, b, preferred_element_type=jnp.float32)


def _dg(a, b, dims):
    return jax.lax.dot_general(a, b, (dims, ((), ())),
                               preferred_element_type=jnp.float32)


NSUB = 4
S2 = NSUB * C


def _scan_body(x_ref, nw_ref, wq_ref, wk_ref, wv_ref, wg_ref, wsm_ref,
               wcv_ref, o_ref, rl_ref, s0_ref, s1_ref, s2_ref, s3_ref,
               hist_ref):
    c = pl.program_id(0)
    srefs = (s0_ref, s1_ref, s2_ref, s3_ref)

    @pl.when(c == 0)
    def _():
        for sr in srefs:
            sr[...] = jnp.zeros_like(sr)
        hist_ref[...] = jnp.zeros_like(hist_ref)

    x = x_ref[...]
    xn = x * jax.lax.rsqrt(jnp.mean(x * x, axis=-1, keepdims=True) + EPS)
    xn = xn * nw_ref[...]
    raw = jnp.concatenate(
        [_f32dot(xn, wq_ref[...]), _f32dot(xn, wk_ref[...]),
         _f32dot(xn, wv_ref[...])], axis=1)
    gate = _f32dot(xn, wg_ref[...])
    small = _f32dot(xn, wsm_ref[...])
    rl_ref[...] = small
    wcv = wcv_ref[...]

    xp = jnp.concatenate([hist_ref[...], raw], axis=0)
    y = (xp[0:S2] * wcv[0:1] + xp[1:S2 + 1] * wcv[1:2]
         + xp[2:S2 + 2] * wcv[2:3] + xp[3:S2 + 3] * wcv[3:4])
    y = y * jax.nn.sigmoid(y)
    hist_ref[...] = raw[S2 - 3:S2, :]

    qr = y[:, 0:256]
    kr = y[:, 256:512]
    v = y[:, 512:768]

    blk = ((jax.lax.broadcasted_iota(jnp.int32, (256, H), 0) // DK)
           == jax.lax.broadcasted_iota(jnp.int32, (256, H), 1)
           ).astype(jnp.float32)
    blkT = ((jax.lax.broadcasted_iota(jnp.int32, (H, 256), 1) // DK)
            == jax.lax.broadcasted_iota(jnp.int32, (H, 256), 0)
            ).astype(jnp.float32)
    qs = jax.lax.rsqrt(_f32dot(qr * qr, blk) + 1e-6)
    ks = jax.lax.rsqrt(_f32dot(kr * kr, blk) + 1e-6)
    q = qr * _f32dot(qs, blkT)
    k = kr * _f32dot(ks, blkT)

    beta = jax.nn.sigmoid(small[:, 0:M])
    g = -jax.nn.softplus(small[:, M:2 * M])
    rl = small[:, 2 * M:3 * M]

    m_iota = jax.lax.broadcasted_iota(jnp.int32, (S2, M), 1)
    mx1 = jnp.max(rl, axis=1, keepdims=True)
    idx1 = jnp.min(jnp.where(rl == mx1, m_iota, M), axis=1, keepdims=True)
    one1 = (m_iota == idx1)
    rl2 = jnp.where(one1, -1e30, rl)
    mx2 = jnp.max(rl2, axis=1, keepdims=True)
    idx2 = jnp.min(jnp.where(rl2 == mx2, m_iota, M), axis=1, keepdims=True)
    one2 = (m_iota == idx2)
    e2 = jnp.exp(mx2 - mx1)
    z = 1.0 + e2
    wfull = (one1.astype(jnp.float32) + one2.astype(jnp.float32) * e2) / z
    ind = (one1 | one2).astype(jnp.float32)

    rep = ((jax.lax.broadcasted_iota(jnp.int32, (H, MH), 1) % H)
           == jax.lax.broadcasted_iota(jnp.int32, (H, MH), 0)
           ).astype(jnp.float32)
    mem = ((jax.lax.broadcasted_iota(jnp.int32, (M, MH), 1) // H)
           == jax.lax.broadcasted_iota(jnp.int32, (M, MH), 0)
           ).astype(jnp.float32)
    b16 = _f32dot(beta, rep) * _f32dot(wfull, mem)
    la16 = _f32dot(g, rep) * _f32dot(ind, mem)

    ii2 = jax.lax.broadcasted_iota(jnp.int32, (S2, S2), 0)
    jj2 = jax.lax.broadcasted_iota(jnp.int32, (S2, S2), 1)
    same = (ii2 // C) == (jj2 // C)
    trilb = ((ii2 >= jj2) & same).astype(jnp.float32)
    triub = ((ii2 <= jj2) & same).astype(jnp.float32)
    L16 = _f32dot(trilb, la16)
    LT = _dg(la16, triub, ((0,), (0,)))

    ii = jax.lax.broadcasted_iota(jnp.int32, (C, C), 0)
    jj = jax.lax.broadcasted_iota(jnp.int32, (C, C), 1)
    low = ii >= jj
    lows = ii > jj

    out = []
    for s in range(NSUB):
        r0 = s * C
        Khs, Qhs, KKs, QKs = [], [], [], []
        Ns, Zs, Dms, eLcs, wms, Lcs = {}, {}, {}, {}, {}, {}
        for h in range(H):
            Kh = k[r0:r0 + C, h * DK:(h + 1) * DK]
            Qh = q[r0:r0 + C, h * DK:(h + 1) * DK]
            Vh = v[r0:r0 + C, h * DV:(h + 1) * DV]
            KK = _dg(Kh, Kh, ((1,), (1,)))
            QK = _dg(Qh, Kh, ((1,), (1,)))
            Khs.append(Kh)
            Qhs.append(Qh)
            KKs.append(KK)
            QKs.append(QK)
            for m in range(M):
                mh = m * H + h
                Lc = L16[r0:r0 + C, mh:mh + 1]
                Lr = LT[mh:mh + 1, r0:r0 + C]
                bc = b16[r0:r0 + C, mh:mh + 1]
                eLc = jnp.exp(Lc)
                Dm = jnp.exp(jnp.where(low, Lc - Lr, -1e30))
                Ns[mh] = jnp.where(lows, (-bc) * Dm * KK, 0.0)
                Zs[mh] = jnp.concatenate([bc * Vh, Kh * (bc * eLc)],
                                         axis=1)
                Dms[mh] = Dm
                eLcs[mh] = eLc
                wms[mh] = wfull[r0:r0 + C, m:m + 1]
                Lcs[mh] = Lc
        for mh in Ns:
            Zs[mh] = Zs[mh] + _f32dot(Ns[mh], Zs[mh])
        for _ in range(LEV - 1):
            for mh in Ns:
                Ns[mh] = _f32dot(Ns[mh], Ns[mh])
            for mh in Ns:
                Zs[mh] = Zs[mh] + _f32dot(Ns[mh], Zs[mh])
        ohs = []
        for h in range(H):
            S0 = srefs[h][...]
            QS = _f32dot(Qhs[h], S0)
            oh = jnp.zeros((C, DV), jnp.float32)
            Snew = []
            for m in range(M):
                mh = m * H + h
                U = Zs[mh][:, 0:DV] - _f32dot(Zs[mh][:, DV:DV + DK],
                                              S0[:, m * DV:(m + 1) * DV])
                oh = oh + wms[mh] * (eLcs[mh] * QS[:, m * DV:(m + 1) * DV]
                                     + _f32dot(Dms[mh] * QKs[h], U))
                LCs = L16[r0 + C - 1:r0 + C, mh:mh + 1]
                Kdec = Khs[h] * jnp.exp(LCs - Lcs[mh])
                Snew.append(jnp.exp(LCs) * S0[:, m * DV:(m + 1) * DV]
                            + _dg(Kdec, U, ((0,), (0,))))
            srefs[h][...] = jnp.concatenate(Snew, axis=1)
            ohs.append(oh)

        o = jnp.concatenate(ohs, axis=1)
        ms = _f32dot(o * o, blk) * (1.0 / DV)
        o = o * _f32dot(jax.lax.rsqrt(ms + EPS), blkT)
        gt = gate[r0:r0 + C]
        out.append(o * (gt * jax.nn.sigmoid(gt)))
    o_ref[...] = jnp.concatenate(out, axis=0)


def _mom_scan(x, norm_w, ap):
    T = x.shape[0]
    wsm = jnp.concatenate(
        [ap['w_beta'], ap['w_g'], ap['w_router'],
         jnp.zeros((D, 128 - 3 * M), jnp.float32)], axis=1)
    wcv = jnp.concatenate(
        [ap['q_conv'], ap['k_conv'], ap['v_conv']], axis=0).T
    khd = H * DK
    cst = pl.BlockSpec((D, khd), lambda c: (0, 0))
    og, rl = pl.pallas_call(
        _scan_body,
        grid=(T // S2,),
        in_specs=[
            pl.BlockSpec((S2, D), lambda c: (c, 0)),
            pl.BlockSpec((1, D), lambda c: (0, 0)),
            cst, cst, cst, cst,
            pl.BlockSpec((D, 128), lambda c: (0, 0)),
            pl.BlockSpec((KC, F3), lambda c: (0, 0)),
        ],
        out_specs=[
            pl.BlockSpec((S2, H * DV), lambda c: (c, 0)),
            pl.BlockSpec((S2, 128), lambda c: (c, 0)),
        ],
        out_shape=[
            jax.ShapeDtypeStruct((T, H * DV), jnp.float32),
            jax.ShapeDtypeStruct((T, 128), jnp.float32),
        ],
        scratch_shapes=[
            pltpu.VMEM((DK, M * DV), jnp.float32),
            pltpu.VMEM((DK, M * DV), jnp.float32),
            pltpu.VMEM((DK, M * DV), jnp.float32),
            pltpu.VMEM((DK, M * DV), jnp.float32),
            pltpu.VMEM((3, F3), jnp.float32),
        ],
    )(x, norm_w.reshape(1, D), ap['wq'], ap['wk'], ap['wv'], ap['w_gate'],
      wsm, wcv)
    return og, rl[:, 2 * M:3 * M]


def _mlp_body(og_ref, wo_ref, res_ref, nw_ref, wg_ref, wu_ref, wd_ref,
              out_ref, xn_ref):
    i = pl.program_id(1)

    @pl.when(i == 0)
    def _():
        r = _f32dot(og_ref[...], wo_ref[...]) + res_ref[...]
        xn = r * jax.lax.rsqrt(jnp.mean(r * r, axis=-1, keepdims=True) + EPS)
        xn_ref[...] = xn * nw_ref[...]
        out_ref[...] = r

    xn = xn_ref[...]
    a = _f32dot(xn, wg_ref[...])
    b = _f32dot(xn, wu_ref[...])
    out_ref[...] += _f32dot(a * jax.nn.sigmoid(a) * b, wd_ref[...])


def _mlp(og, wo_s, res, norm_w, mp):
    T = og.shape[0]
    bm = 256
    bi = 512
    return pl.pallas_call(
        _mlp_body,
        grid=(T // bm, I // bi),
        in_specs=[
            pl.BlockSpec((bm, H * DV), lambda t, i: (t, 0)),
            pl.BlockSpec((H * DV, D), lambda t, i: (0, 0)),
            pl.BlockSpec((bm, D), lambda t, i: (t, 0)),
            pl.BlockSpec((1, D), lambda t, i: (0, 0)),
            pl.BlockSpec((D, bi), lambda t, i: (0, i)),
            pl.BlockSpec((D, bi), lambda t, i: (0, i)),
            pl.BlockSpec((bi, D), lambda t, i: (i, 0)),
        ],
        out_specs=pl.BlockSpec((bm, D), lambda t, i: (t, 0)),
        out_shape=jax.ShapeDtypeStruct((T, D), jnp.float32),
        scratch_shapes=[pltpu.VMEM((bm, D), jnp.float32)],
    )(og, wo_s, res, norm_w.reshape(1, D), mp['wg'], mp['wu'], mp['wd'])


def _wo_scaled(ap):
    return ap['wo'] * jnp.tile(ap['o_norm_w'], H)[:, None]


def kernel(query, keyval, params):
    p = params
    x0 = query[0]
    kv = keyval[0]
    a1, a2 = p['qa_attn'], p['qkv_attn']

    OG1, q_rl = _mom_scan(x0, p['qa_attn_norm'], a1)
    out1 = _mlp(OG1, _wo_scaled(a1), x0, p['qa_mlp_norm'], p['qa_mlp'])

    qkv = jnp.concatenate([kv, out1], axis=0)
    OG2, kv_rl = _mom_scan(qkv, p['qkv_norm'], a2)
    out2 = _mlp(OG2[kv.shape[0]:], _wo_scaled(a2), out1, p['ffn_norm'],
                p['ffn'])

    return out2[None], q_rl[None], kv_rl[None]


# speedup vs baseline: 1.3039x; 1.3039x over previous
import functools

import jax
import jax.numpy as jnp
from jax.experimental import pallas as pl
from jax.experimental.pallas import tpu as pltpu

D = 1024
H = 4
DK = 64
DV = 64
M = 4
KC = 4
I = 2048
EPS = 1e-6
C = 128
LEV = 7
MH = M * H
PW = 1152
F3 = 3 * H * DK


def _f32dot(a, b):
    return jnp.dot(a, b, preferred_element_type=jnp.float32)


def _dg(a, b, dims):
    return jax.lax.dot_general(a, b, (dims, ((), ())),
                               preferred_element_type=jnp.float32)


NSUB = 2
S2 = NSUB * C


def _scan_body(x_ref, nw_ref, wq_ref, wk_ref, wv_ref, wg_ref, wsm_ref,
               wcv_ref, o_ref, rl_ref, s0_ref, s1_ref, s2_ref, s3_ref,
               hist_ref):
    c = pl.program_id(0)
    srefs = (s0_ref, s1_ref, s2_ref, s3_ref)

    @pl.when(c == 0)
    def _():
        for sr in srefs:
            sr[...] = jnp.zeros_like(sr)
        hist_ref[...] = jnp.zeros_like(hist_ref)

    x = x_ref[...]
    xn = x * jax.lax.rsqrt(jnp.mean(x * x, axis=-1, keepdims=True) + EPS)
    xn = xn * nw_ref[...]
    raw = jnp.concatenate(
        [_f32dot(xn, wq_ref[...]), _f32dot(xn, wk_ref[...]),
         _f32dot(xn, wv_ref[...])], axis=1)
    gate = _f32dot(xn, wg_ref[...])
    small = _f32dot(xn, wsm_ref[...])
    rl_ref[...] = small
    wcv = wcv_ref[...]

    xp = jnp.concatenate([hist_ref[...], raw], axis=0)
    y = (xp[0:S2] * wcv[0:1] + xp[1:S2 + 1] * wcv[1:2]
         + xp[2:S2 + 2] * wcv[2:3] + xp[3:S2 + 3] * wcv[3:4])
    y = y * jax.nn.sigmoid(y)
    hist_ref[...] = raw[S2 - 3:S2, :]

    qr = y[:, 0:256]
    kr = y[:, 256:512]
    v = y[:, 512:768]

    blk = ((jax.lax.broadcasted_iota(jnp.int32, (256, H), 0) // DK)
           == jax.lax.broadcasted_iota(jnp.int32, (256, H), 1)
           ).astype(jnp.float32)
    blkT = ((jax.lax.broadcasted_iota(jnp.int32, (H, 256), 1) // DK)
            == jax.lax.broadcasted_iota(jnp.int32, (H, 256), 0)
            ).astype(jnp.float32)
    qs = jax.lax.rsqrt(_f32dot(qr * qr, blk) + 1e-6)
    ks = jax.lax.rsqrt(_f32dot(kr * kr, blk) + 1e-6)
    q = qr * _f32dot(qs, blkT)
    k = kr * _f32dot(ks, blkT)

    beta = jax.nn.sigmoid(small[:, 0:M])
    g = -jax.nn.softplus(small[:, M:2 * M])
    rl = small[:, 2 * M:3 * M]

    m_iota = jax.lax.broadcasted_iota(jnp.int32, (S2, M), 1)
    mx1 = jnp.max(rl, axis=1, keepdims=True)
    idx1 = jnp.min(jnp.where(rl == mx1, m_iota, M), axis=1, keepdims=True)
    one1 = (m_iota == idx1)
    rl2 = jnp.where(one1, -1e30, rl)
    mx2 = jnp.max(rl2, axis=1, keepdims=True)
    idx2 = jnp.min(jnp.where(rl2 == mx2, m_iota, M), axis=1, keepdims=True)
    one2 = (m_iota == idx2)
    e2 = jnp.exp(mx2 - mx1)
    z = 1.0 + e2
    wfull = (one1.astype(jnp.float32) + one2.astype(jnp.float32) * e2) / z
    ind = (one1 | one2).astype(jnp.float32)

    rep = ((jax.lax.broadcasted_iota(jnp.int32, (H, MH), 1) % H)
           == jax.lax.broadcasted_iota(jnp.int32, (H, MH), 0)
           ).astype(jnp.float32)
    mem = ((jax.lax.broadcasted_iota(jnp.int32, (M, MH), 1) // H)
           == jax.lax.broadcasted_iota(jnp.int32, (M, MH), 0)
           ).astype(jnp.float32)
    b16 = _f32dot(beta, rep) * _f32dot(wfull, mem)
    la16 = _f32dot(g, rep) * _f32dot(ind, mem)

    ii2 = jax.lax.broadcasted_iota(jnp.int32, (S2, S2), 0)
    jj2 = jax.lax.broadcasted_iota(jnp.int32, (S2, S2), 1)
    same = (ii2 // C) == (jj2 // C)
    trilb = ((ii2 >= jj2) & same).astype(jnp.float32)
    triub = ((ii2 <= jj2) & same).astype(jnp.float32)
    L16 = _f32dot(trilb, la16)
    LT = _dg(la16, triub, ((0,), (0,)))

    ii = jax.lax.broadcasted_iota(jnp.int32, (C, C), 0)
    jj = jax.lax.broadcasted_iota(jnp.int32, (C, C), 1)
    low = ii >= jj
    lows = ii > jj

    out = []
    for s in range(NSUB):
        r0 = s * C
        Khs, Qhs, KKs, QKs = [], [], [], []
        Ns, Zs, Dms, eLcs, wms, Lcs = {}, {}, {}, {}, {}, {}
        for h in range(H):
            Kh = k[r0:r0 + C, h * DK:(h + 1) * DK]
            Qh = q[r0:r0 + C, h * DK:(h + 1) * DK]
            Vh = v[r0:r0 + C, h * DV:(h + 1) * DV]
            KK = _dg(Kh, Kh, ((1,), (1,)))
            QK = _dg(Qh, Kh, ((1,), (1,)))
            Khs.append(Kh)
            Qhs.append(Qh)
            KKs.append(KK)
            QKs.append(QK)
            for m in range(M):
                mh = m * H + h
                Lc = L16[r0:r0 + C, mh:mh + 1]
                Lr = LT[mh:mh + 1, r0:r0 + C]
                bc = b16[r0:r0 + C, mh:mh + 1]
                eLc = jnp.exp(Lc)
                Dm = jnp.exp(jnp.where(low, Lc - Lr, -1e30))
                Ns[mh] = jnp.where(lows, (-bc) * Dm * KK, 0.0)
                Zs[mh] = jnp.concatenate([bc * Vh, Kh * (bc * eLc)],
                                         axis=1)
                Dms[mh] = Dm
                eLcs[mh] = eLc
                wms[mh] = wfull[r0:r0 + C, m:m + 1]
                Lcs[mh] = Lc
        for mh in Ns:
            Zs[mh] = Zs[mh] + _f32dot(Ns[mh], Zs[mh])
        for _ in range(LEV - 1):
            for mh in Ns:
                Ns[mh] = _f32dot(Ns[mh], Ns[mh])
            for mh in Ns:
                Zs[mh] = Zs[mh] + _f32dot(Ns[mh], Zs[mh])
        ohs = []
        for h in range(H):
            S0 = srefs[h][...]
            QS = _f32dot(Qhs[h], S0)
            oh = jnp.zeros((C, DV), jnp.float32)
            Snew = []
            for m in range(M):
                mh = m * H + h
                U = Zs[mh][:, 0:DV] - _f32dot(Zs[mh][:, DV:DV + DK],
                                              S0[:, m * DV:(m + 1) * DV])
                oh = oh + wms[mh] * (eLcs[mh] * QS[:, m * DV:(m + 1) * DV]
                                     + _f32dot(Dms[mh] * QKs[h], U))
                LCs = L16[r0 + C - 1:r0 + C, mh:mh + 1]
                Kdec = Khs[h] * jnp.exp(LCs - Lcs[mh])
                Snew.append(jnp.exp(LCs) * S0[:, m * DV:(m + 1) * DV]
                            + _dg(Kdec, U, ((0,), (0,))))
            srefs[h][...] = jnp.concatenate(Snew, axis=1)
            ohs.append(oh)

        o = jnp.concatenate(ohs, axis=1)
        ms = _f32dot(o * o, blk) * (1.0 / DV)
        o = o * _f32dot(jax.lax.rsqrt(ms + EPS), blkT)
        gt = gate[r0:r0 + C]
        out.append(o * (gt * jax.nn.sigmoid(gt)))
    o_ref[...] = jnp.concatenate(out, axis=0)


def _mom_scan(x, norm_w, ap):
    T = x.shape[0]
    wsm = jnp.concatenate(
        [ap['w_beta'], ap['w_g'], ap['w_router'],
         jnp.zeros((D, 128 - 3 * M), jnp.float32)], axis=1)
    wcv = jnp.concatenate(
        [ap['q_conv'], ap['k_conv'], ap['v_conv']], axis=0).T
    khd = H * DK
    cst = pl.BlockSpec((D, khd), lambda c: (0, 0))
    og, rl = pl.pallas_call(
        _scan_body,
        grid=(T // S2,),
        in_specs=[
            pl.BlockSpec((S2, D), lambda c: (c, 0)),
            pl.BlockSpec((1, D), lambda c: (0, 0)),
            cst, cst, cst, cst,
            pl.BlockSpec((D, 128), lambda c: (0, 0)),
            pl.BlockSpec((KC, F3), lambda c: (0, 0)),
        ],
        out_specs=[
            pl.BlockSpec((S2, H * DV), lambda c: (c, 0)),
            pl.BlockSpec((S2, 128), lambda c: (c, 0)),
        ],
        out_shape=[
            jax.ShapeDtypeStruct((T, H * DV), jnp.float32),
            jax.ShapeDtypeStruct((T, 128), jnp.float32),
        ],
        scratch_shapes=[
            pltpu.VMEM((DK, M * DV), jnp.float32),
            pltpu.VMEM((DK, M * DV), jnp.float32),
            pltpu.VMEM((DK, M * DV), jnp.float32),
            pltpu.VMEM((DK, M * DV), jnp.float32),
            pltpu.VMEM((3, F3), jnp.float32),
        ],
    )(x, norm_w.reshape(1, D), ap['wq'], ap['wk'], ap['wv'], ap['w_gate'],
      wsm, wcv)
    return og, rl[:, 2 * M:3 * M]


def _mlp_body(og_ref, wo_ref, res_ref, nw_ref, wg_ref, wu_ref, wd_ref,
              out_ref, xn_ref):
    i = pl.program_id(1)

    @pl.when(i == 0)
    def _():
        r = _f32dot(og_ref[...], wo_ref[...]) + res_ref[...]
        xn = r * jax.lax.rsqrt(jnp.mean(r * r, axis=-1, keepdims=True) + EPS)
        xn_ref[...] = xn * nw_ref[...]
        out_ref[...] = r

    xn = xn_ref[...]
    a = _f32dot(xn, wg_ref[...])
    b = _f32dot(xn, wu_ref[...])
    out_ref[...] += _f32dot(a * jax.nn.sigmoid(a) * b, wd_ref[...])


def _mlp(og, wo_s, res, norm_w, mp):
    T = og.shape[0]
    bm = 256
    bi = 512
    return pl.pallas_call(
        _mlp_body,
        grid=(T // bm, I // bi),
        in_specs=[
            pl.BlockSpec((bm, H * DV), lambda t, i: (t, 0)),
            pl.BlockSpec((H * DV, D), lambda t, i: (0, 0)),
            pl.BlockSpec((bm, D), lambda t, i: (t, 0)),
            pl.BlockSpec((1, D), lambda t, i: (0, 0)),
            pl.BlockSpec((D, bi), lambda t, i: (0, i)),
            pl.BlockSpec((D, bi), lambda t, i: (0, i)),
            pl.BlockSpec((bi, D), lambda t, i: (i, 0)),
        ],
        out_specs=pl.BlockSpec((bm, D), lambda t, i: (t, 0)),
        out_shape=jax.ShapeDtypeStruct((T, D), jnp.float32),
        scratch_shapes=[pltpu.VMEM((bm, D), jnp.float32)],
    )(og, wo_s, res, norm_w.reshape(1, D), mp['wg'], mp['wu'], mp['wd'])


def _wo_scaled(ap):
    return ap['wo'] * jnp.tile(ap['o_norm_w'], H)[:, None]


def kernel(query, keyval, params):
    p = params
    x0 = query[0]
    kv = keyval[0]
    a1, a2 = p['qa_attn'], p['qkv_attn']

    OG1, q_rl = _mom_scan(x0, p['qa_attn_norm'], a1)
    out1 = _mlp(OG1, _wo_scaled(a1), x0, p['qa_mlp_norm'], p['qa_mlp'])

    qkv = jnp.concatenate([kv, out1], axis=0)
    OG2, kv_rl = _mom_scan(qkv, p['qkv_norm'], a2)
    out2 = _mlp(OG2[kv.shape[0]:], _wo_scaled(a2), out1, p['ffn_norm'],
                p['ffn'])

    return out2[None], q_rl[None], kv_rl[None]
